# baseline (speedup 1.0000x reference)
import functools

import jax
import jax.numpy as jnp
from jax import lax
from jax.experimental import pallas as pl
from jax.experimental.pallas import tpu as pltpu
from jax.experimental.pallas import tpu_sc as plsc

_NQ = 16384
_NK = 8192
_LOSS_WEIGHT = 1.0

_NSC = 4096
_NW = 32
_QPW = _NSC // _NW
_QB = 8
_L = 16
_QBLK = 1024


def _sc_body(q_hbm, k_hbm, out_hbm, kv, qv, outv):
    wid = lax.axis_index("s") * 2 + lax.axis_index("c")
    pltpu.sync_copy(k_hbm, kv.at[pl.ds(0, 3)])
    pltpu.sync_copy(q_hbm.at[wid], qv)

    iota = lax.iota(jnp.int32, _L)
    big = jnp.full((_L,), jnp.int32(1 << 30))
    ninf = jnp.full((_L,), -jnp.inf, jnp.float32)
    zero_i = jnp.full((_L,), jnp.int32(0))
    perms = [iota ^ k for k in (1, 2, 4, 8)]
    _dnums = lax.GatherDimensionNumbers(
        offset_dims=(), collapsed_slice_dims=(0,), start_index_map=(0,))

    def _shuf(v, perm):
        return lax.gather(v, perm[:, None], _dnums, (1,),
                          mode=lax.GatherScatterMode.PROMISE_IN_BOUNDS)

    def _bfly(v, op):
        for perm in perms:
            v = op(v, _shuf(v, perm))
        return v

    def stage(kb, c):
        kx = kv[0, pl.ds(kb * _L, _L)]
        ky = kv[1, pl.ds(kb * _L, _L)]
        kz = kv[2, pl.ds(kb * _L, _L)]
        kv[3, pl.ds(kb * _L, _L)] = -0.5 * (kx * kx + ky * ky + kz * kz)
        return c

    lax.fori_loop(0, _NK // _L, stage, 0)

    def outer(qb, acc):
        qxv = qv[0, pl.ds(qb * _L, _L)]
        qyv = qv[1, pl.ds(qb * _L, _L)]
        qzv = qv[2, pl.ds(qb * _L, _L)]
        for p in range(2):
            qs = []
            for u in range(_QB):
                lane = p * _QB + u
                qs.append((jnp.broadcast_to(qxv[lane], (_L,)),
                           jnp.broadcast_to(qyv[lane], (_L,)),
                           jnp.broadcast_to(qzv[lane], (_L,))))

            def inner(kb, carry):
                bvs, bis = carry
                base = kb * _L
                kx = kv[0, pl.ds(base, _L)]
                ky = kv[1, pl.ds(base, _L)]
                kz = kv[2, pl.ds(base, _L)]
                tv = kv[3, pl.ds(base, _L)]
                idxv = iota + base
                nbvs, nbis = [], []
                for u in range(_QB):
                    qx, qy, qz = qs[u]
                    s = qx * kx + tv
                    s = s + qy * ky
                    s = s + qz * kz
                    gt = s > bvs[u]
                    nbvs.append(jnp.where(gt, s, bvs[u]))
                    nbis.append(jnp.where(gt, idxv, bis[u]))
                return tuple(nbvs), tuple(nbis)

            bvs, bis = lax.fori_loop(
                0, _NK // _L, inner,
                (tuple(ninf for _ in range(_QB)),
                 tuple(zero_i for _ in range(_QB))),
                unroll=2)
            for u in range(_QB):
                m = _bfly(bvs[u], jnp.maximum)
                cand = jnp.where(bvs[u] == m, bis[u], big)
                bi = _bfly(cand, jnp.minimum)
                acc = acc + bi.astype(jnp.float32)
        return acc

    acc = lax.fori_loop(0, _QPW // 16, outer,
                        jnp.zeros((_L,), jnp.float32))
    outv[...] = acc
    pltpu.sync_copy(outv, out_hbm.at[wid])


def _sc_sums(q_t, k_t):
    mesh = plsc.VectorSubcoreMesh(core_axis_name="c", subcore_axis_name="s",
                                  num_cores=2, num_subcores=16)
    f = pl.kernel(
        _sc_body,
        mesh=mesh,
        out_type=jax.ShapeDtypeStruct((_NW, _L), jnp.float32),
        scratch_types=[
            pltpu.VMEM((4, _NK), jnp.float32),
            pltpu.VMEM((3, _QPW), jnp.float32),
            pltpu.VMEM((_L,), jnp.float32),
        ],
    )
    return f(q_t, k_t)


def _tc_body(q_ref, kt_ref, out_ref):
    val = jnp.dot(q_ref[...], kt_ref[...],
                  preferred_element_type=jnp.float32)
    m = jnp.max(val, axis=1, keepdims=True)
    ids = jax.lax.broadcasted_iota(jnp.int32, val.shape, 1)
    idx = jnp.min(jnp.where(val == m, ids, _NK), axis=1)
    out_ref[0, 0, :] = jnp.broadcast_to(
        jnp.sum(idx.astype(jnp.float32)), (128,))


def _tc_sums(q_pad, kt_pad):
    grid = q_pad.shape[0] // _QBLK
    return pl.pallas_call(
        _tc_body,
        grid=(grid,),
        in_specs=[
            pl.BlockSpec((_QBLK, 8), lambda i: (i, 0)),
            pl.BlockSpec((8, _NK), lambda i: (0, 0)),
        ],
        out_specs=pl.BlockSpec((1, 1, 128), lambda i: (i, 0, 0)),
        out_shape=jax.ShapeDtypeStruct((grid, 1, 128), jnp.float32),
    )(q_pad, kt_pad)


def kernel(adv_pc, ori_pc):
    k = ori_pc[:, :3]

    total = jnp.float32(0.0)
    if _NSC > 0:
        assert _NSC <= 8192
        q_sc = adv_pc[:_NSC, :3].reshape(_NW, _QPW, 3).transpose(0, 2, 1)
        total = total + jnp.sum(_sc_sums(q_sc, k.T)[:, 0])
    if _NSC < _NQ:
        bias = -0.5 * jnp.sum(k * k, axis=1, keepdims=True)
        q_tc = jnp.concatenate([adv_pc[_NSC:, :3], k], axis=0)
        ones = jnp.ones((q_tc.shape[0], 1), jnp.float32)
        q_pad = jnp.pad(jnp.concatenate([q_tc, ones], axis=1),
                        ((0, 0), (0, 4)))
        kt_pad = jnp.pad(jnp.concatenate([k, bias], axis=1).T,
                         ((0, 4), (0, 0)))
        total = total + jnp.sum(_tc_sums(q_pad, kt_pad)[:, 0, 0])
    return (total / jnp.float32(_NK)) * _LOSS_WEIGHT


# speedup vs baseline: 1.1071x; 1.1071x over previous
import functools

import jax
import jax.numpy as jnp
from jax import lax
from jax.experimental import pallas as pl
from jax.experimental.pallas import tpu as pltpu
from jax.experimental.pallas import tpu_sc as plsc

_NQ = 16384
_NK = 8192
_LOSS_WEIGHT = 1.0

_NSC = 4096
_NW = 32
_QPW = _NSC // _NW
_QB = 8
_L = 16
_QBLK = 1024


def _sc_body(q_hbm, k_hbm, out_hbm, kv, qv, outv):
    wid = lax.axis_index("s") * 2 + lax.axis_index("c")
    pltpu.sync_copy(k_hbm, kv.at[pl.ds(0, 3)])
    pltpu.sync_copy(q_hbm.at[wid], qv)

    iota = lax.iota(jnp.int32, _L)
    big = jnp.full((_L,), jnp.int32(1 << 30))
    ninf = jnp.full((_L,), -jnp.inf, jnp.float32)
    zero_i = jnp.full((_L,), jnp.int32(0))
    perms = [iota ^ k for k in (1, 2, 4, 8)]
    _dnums = lax.GatherDimensionNumbers(
        offset_dims=(), collapsed_slice_dims=(0,), start_index_map=(0,))

    def _shuf(v, perm):
        return lax.gather(v, perm[:, None], _dnums, (1,),
                          mode=lax.GatherScatterMode.PROMISE_IN_BOUNDS)

    def _bfly(v, op):
        for perm in perms:
            v = op(v, _shuf(v, perm))
        return v

    def stage(kb, c):
        kx = kv[0, pl.ds(kb * _L, _L)]
        ky = kv[1, pl.ds(kb * _L, _L)]
        kz = kv[2, pl.ds(kb * _L, _L)]
        kv[3, pl.ds(kb * _L, _L)] = -0.5 * (kx * kx + ky * ky + kz * kz)
        return c

    lax.fori_loop(0, _NK // _L, stage, 0)

    def outer(qb, acc):
        qxv = qv[0, pl.ds(qb * _L, _L)]
        qyv = qv[1, pl.ds(qb * _L, _L)]
        qzv = qv[2, pl.ds(qb * _L, _L)]
        for p in range(2):
            qs = []
            for u in range(_QB):
                lane = p * _QB + u
                qs.append((jnp.broadcast_to(qxv[lane], (_L,)),
                           jnp.broadcast_to(qyv[lane], (_L,)),
                           jnp.broadcast_to(qzv[lane], (_L,))))

            def inner(kb, carry):
                bvs, bis = carry
                base = kb * _L
                kx = kv[0, pl.ds(base, _L)]
                ky = kv[1, pl.ds(base, _L)]
                kz = kv[2, pl.ds(base, _L)]
                tv = kv[3, pl.ds(base, _L)]
                idxv = iota + base
                nbvs, nbis = [], []
                for u in range(_QB):
                    qx, qy, qz = qs[u]
                    s = qx * kx + tv
                    s = s + qy * ky
                    s = s + qz * kz
                    gt = s > bvs[u]
                    nbvs.append(jnp.where(gt, s, bvs[u]))
                    nbis.append(jnp.where(gt, idxv, bis[u]))
                return tuple(nbvs), tuple(nbis)

            bvs, bis = lax.fori_loop(
                0, _NK // _L, inner,
                (tuple(ninf for _ in range(_QB)),
                 tuple(zero_i for _ in range(_QB))))
            for u in range(_QB):
                m = _bfly(bvs[u], jnp.maximum)
                cand = jnp.where(bvs[u] == m, bis[u], big)
                bi = _bfly(cand, jnp.minimum)
                acc = acc + bi.astype(jnp.float32)
        return acc

    acc = lax.fori_loop(0, _QPW // 16, outer,
                        jnp.zeros((_L,), jnp.float32))
    outv[...] = acc
    pltpu.sync_copy(outv, out_hbm.at[wid])


def _sc_sums(q_t, k_t):
    mesh = plsc.VectorSubcoreMesh(core_axis_name="c", subcore_axis_name="s",
                                  num_cores=2, num_subcores=16)
    f = pl.kernel(
        _sc_body,
        mesh=mesh,
        out_type=jax.ShapeDtypeStruct((_NW, _L), jnp.float32),
        scratch_types=[
            pltpu.VMEM((4, _NK), jnp.float32),
            pltpu.VMEM((3, _QPW), jnp.float32),
            pltpu.VMEM((_L,), jnp.float32),
        ],
    )
    return f(q_t, k_t)


def _tc_body(q_ref, kt_ref, out_ref):
    val = jnp.dot(q_ref[...], kt_ref[...],
                  preferred_element_type=jnp.float32)
    m = jnp.max(val, axis=1, keepdims=True)
    ids = jax.lax.broadcasted_iota(jnp.int32, val.shape, 1)
    idx = jnp.min(jnp.where(val == m, ids, _NK), axis=1)
    out_ref[0, 0, :] = jnp.broadcast_to(
        jnp.sum(idx.astype(jnp.float32)), (128,))


def _tc_sums(q_pad, kt_pad):
    grid = q_pad.shape[0] // _QBLK
    return pl.pallas_call(
        _tc_body,
        grid=(grid,),
        in_specs=[
            pl.BlockSpec((_QBLK, 8), lambda i: (i, 0)),
            pl.BlockSpec((8, _NK), lambda i: (0, 0)),
        ],
        out_specs=pl.BlockSpec((1, 1, 128), lambda i: (i, 0, 0)),
        out_shape=jax.ShapeDtypeStruct((grid, 1, 128), jnp.float32),
    )(q_pad, kt_pad)


def kernel(adv_pc, ori_pc):
    k = ori_pc[:, :3]

    total = jnp.float32(0.0)
    if _NSC > 0:
        assert _NSC <= 8192
        q_sc = adv_pc[:_NSC, :3].reshape(_NW, _QPW, 3).transpose(0, 2, 1)
        total = total + jnp.sum(_sc_sums(q_sc, k.T)[:, 0])
    if _NSC < _NQ:
        bias = -0.5 * jnp.sum(k * k, axis=1, keepdims=True)
        q_tc = jnp.concatenate([adv_pc[_NSC:, :3], k], axis=0)
        ones = jnp.ones((q_tc.shape[0], 1), jnp.float32)
        q_pad = jnp.pad(jnp.concatenate([q_tc, ones], axis=1),
                        ((0, 0), (0, 4)))
        kt_pad = jnp.pad(jnp.concatenate([k, bias], axis=1).T,
                         ((0, 4), (0, 0)))
        total = total + jnp.sum(_tc_sums(q_pad, kt_pad)[:, 0, 0])
    return (total / jnp.float32(_NK)) * _LOSS_WEIGHT
